# Initial kernel scaffold; baseline (speedup 1.0000x reference)
import functools
import math

import jax
import jax.numpy as jnp
from jax import lax
from jax.experimental import pallas as pl
from jax.experimental.pallas import tpu as pltpu
from jax.experimental.pallas import tpu_sc as plsc

_NUM_LAYERS = 4
_ALPHA = 0.1
_THETA = 0.5
_NUM_GRAPHS = 16
_EPS_PN = 1e-5
_EPS_BN = 1e-5

_NC = 2
_NS = 16
_NW = _NC * _NS


@functools.partial(jax.jit, static_argnames=("n", "hid", "ch", "k"))
def _sc_segment_sum(h, src3, dst3, zeros_tile, *, n, hid, ch, k):
    rpt = n // _NS

    mesh = plsc.VectorSubcoreMesh(core_axis_name="c", subcore_axis_name="s")

    @functools.partial(
        pl.kernel,
        mesh=mesh,
        out_type=jax.ShapeDtypeStruct((_NC, n, hid), jnp.float32),
        scratch_types=[
            pltpu.VMEM((ch, k), jnp.int32),
            pltpu.VMEM((ch, k), jnp.int32),
            pltpu.VMEM((k, hid), jnp.float32),
            pltpu.VMEM_SHARED((n, hid), jnp.float32),
            pltpu.SemaphoreType.DMA,
        ],
    )
    def body(h_hbm, src_hbm, dst_hbm, z_hbm, out_hbm, src_v, dst_v, rows_v,
             acc_sh, sem):
        cid = lax.axis_index("c")
        sid = lax.axis_index("s")
        wid = sid * _NC + cid
        pltpu.sync_copy(z_hbm, acc_sh.at[pl.ds(sid * rpt, rpt)])
        pltpu.sync_copy(src_hbm.at[wid], src_v)
        pltpu.sync_copy(dst_hbm.at[wid], dst_v)
        plsc.subcore_barrier()

        def step(j, carry):
            pltpu.async_copy(h_hbm.at[src_v.at[j]], rows_v, sem).wait()
            pltpu.sync_copy(rows_v, acc_sh.at[dst_v.at[j]], add=True)
            return carry

        lax.fori_loop(0, ch, step, 0)
        plsc.subcore_barrier()
        pltpu.sync_copy(acc_sh.at[pl.ds(sid * rpt, rpt)],
                        out_hbm.at[cid, pl.ds(sid * rpt, rpt)])

    return body(h, src3, dst3, zeros_tile)


def _dense0_body(x_ref, w_ref, b_ref, o_ref):
    o_ref[...] = jax.nn.relu(
        jnp.dot(x_ref[...], w_ref[...], preferred_element_type=jnp.float32)
        + b_ref[...])


def _layer_body(p_ref, x0_ref, h_ref, w_ref, o_ref, *, alpha, beta, eps):
    agg = p_ref[0] + p_ref[1]
    t = (1.0 - alpha) * agg + alpha * x0_ref[...]
    out = (1.0 - beta) * t + beta * jnp.dot(
        t, w_ref[...], preferred_element_type=jnp.float32)
    h2 = jax.nn.relu(out + h_ref[...])
    h2 = h2 - jnp.mean(h2, axis=0, keepdims=True)
    ms = jnp.mean(jnp.sum(h2 * h2, axis=-1))
    o_ref[...] = h2 / jnp.sqrt(eps + ms)


def _tail_body(h_ref, bat_ref, w1_ref, b1_ref, w2_ref, b2_ref, g_ref, be_ref,
               o_ref, *, ng, eps_bn):
    h = h_ref[...]
    bat = bat_ref[...]
    gmax_rows = []
    gsum_rows = []
    cnt_rows = []
    for g in range(ng):
        m = bat == g
        gmax_rows.append(
            jnp.max(jnp.where(m, h, -jnp.inf), axis=0, keepdims=True))
        gsum_rows.append(
            jnp.sum(jnp.where(m, h, 0.0), axis=0, keepdims=True))
        cnt_rows.append(jnp.sum(m.astype(jnp.float32), axis=0, keepdims=True))
    gmax = jnp.concatenate(gmax_rows, axis=0)
    gsum = jnp.concatenate(gsum_rows, axis=0)
    cnt = jnp.concatenate(cnt_rows, axis=0)
    cnt = jnp.maximum(cnt, 1.0)
    x2 = jnp.concatenate([gmax, gsum / cnt], axis=1)
    z = jax.nn.relu(
        jnp.dot(x2, w1_ref[...], preferred_element_type=jnp.float32)
        + b1_ref[...])
    mu = jnp.mean(z, axis=0, keepdims=True)
    var = jnp.mean((z - mu) ** 2, axis=0, keepdims=True)
    z = (z - mu) / jnp.sqrt(var + eps_bn) * g_ref[...] + be_ref[...]
    logits = (jnp.dot(z, w2_ref[...], preferred_element_type=jnp.float32)
              + b2_ref[...])
    mx = jnp.max(logits, axis=1, keepdims=True)
    lse = mx + jnp.log(jnp.sum(jnp.exp(logits - mx), axis=1, keepdims=True))
    o_ref[...] = logits - lse


def _tc_call(body, out_shape, *args):
    return pl.pallas_call(body, out_shape=out_shape)(*args)


def kernel(x, edge_index, batch, lin0_W, lin0_b, conv_W, lin1_W, lin1_b,
           lin2_W, lin2_b, bn_gamma, bn_beta):
    n, dim = x.shape
    hid = lin0_W.shape[1]
    e = edge_index.shape[1]

    assert e % _NW == 0, e
    epw = e // _NW
    k = 0
    for cand in (128, 112, 96, 80, 64, 48, 32, 16, 8):
        if epw % cand == 0:
            k = cand
            break
    assert k, epw
    ch = epw // k
    assert n % _NS == 0, n

    src3 = edge_index[0].reshape(_NW, ch, k)
    dst3 = edge_index[1].reshape(_NW, ch, k)
    zeros_tile = jnp.zeros((n // _NS, hid), dtype=jnp.float32)
    bat2 = batch.reshape(n, 1)

    x0 = _tc_call(_dense0_body, jax.ShapeDtypeStruct((n, hid), jnp.float32),
                  x, lin0_W, lin0_b.reshape(1, hid))
    h = x0
    for l in range(_NUM_LAYERS):
        beta = float(math.log(_THETA / (l + 1) + 1.0))
        partials = _sc_segment_sum(h, src3, dst3, zeros_tile,
                                   n=n, hid=hid, ch=ch, k=k)
        h = _tc_call(
            functools.partial(_layer_body, alpha=_ALPHA, beta=beta,
                              eps=_EPS_PN),
            jax.ShapeDtypeStruct((n, hid), jnp.float32),
            partials, x0, h, conv_W[l])

    out = _tc_call(
        functools.partial(_tail_body, ng=_NUM_GRAPHS, eps_bn=_EPS_BN),
        jax.ShapeDtypeStruct((_NUM_GRAPHS, lin2_W.shape[1]), jnp.float32),
        h, bat2, lin1_W, lin1_b.reshape(1, -1), lin2_W, lin2_b.reshape(1, -1),
        bn_gamma.reshape(1, -1), bn_beta.reshape(1, -1))
    return out


# speedup vs baseline: 11.7731x; 11.7731x over previous
import functools
import math

import jax
import jax.numpy as jnp
from jax import lax
from jax.experimental import pallas as pl
from jax.experimental.pallas import tpu as pltpu
from jax.experimental.pallas import tpu_sc as plsc

_NUM_LAYERS = 4
_ALPHA = 0.1
_THETA = 0.5
_NUM_GRAPHS = 16
_EPS_PN = 1e-5
_EPS_BN = 1e-5

_NC = 2
_NS = 16
_NW = _NC * _NS


@functools.partial(jax.jit, static_argnames=("npad", "hid", "ch", "k"))
def _sc_segment_sum(h, src1, dst3, zeros_tile, *, npad, hid, ch, k):
    rpt = npad // _NS
    epw = ch * k

    mesh = plsc.VectorSubcoreMesh(core_axis_name="c", subcore_axis_name="s")

    @functools.partial(
        pl.kernel,
        mesh=mesh,
        out_type=jax.ShapeDtypeStruct((_NC, npad, hid), jnp.float32),
        scratch_types=[
            pltpu.VMEM((epw,), jnp.int32),
            pltpu.VMEM((3, 1, k), jnp.int32),
            pltpu.VMEM((k, hid), jnp.float32),
            pltpu.VMEM((k, hid), jnp.float32),
            pltpu.VMEM((k, hid), jnp.float32),
            pltpu.VMEM_SHARED((npad, hid), jnp.float32),
            pltpu.SemaphoreType.DMA,
            pltpu.SemaphoreType.DMA,
            pltpu.SemaphoreType.DMA,
            pltpu.SemaphoreType.DMA,
            pltpu.SemaphoreType.DMA,
            pltpu.SemaphoreType.DMA,
        ],
    )
    def body(h_hbm, src_hbm, dst_hbm, z_hbm, out_hbm, src_v, dst_v, rows0,
             rows1, rows2, acc_sh, gs0, gs1, gs2, ds0, ds1, ds2):
        cid = lax.axis_index("c")
        sid = lax.axis_index("s")
        wid = sid * _NC + cid
        rows = (rows0, rows1, rows2)
        gsem = (gs0, gs1, gs2)
        dsem = (ds0, ds1, ds2)
        pltpu.sync_copy(z_hbm, acc_sh.at[pl.ds(sid * rpt, rpt)])
        pltpu.sync_copy(src_hbm.at[pl.ds(wid * epw, epw)], src_v)
        plsc.subcore_barrier()

        def gather(c, u):
            return pltpu.make_async_copy(
                h_hbm.at[src_v.at[pl.ds(c * k, k)]], rows[u], gsem[u])

        def dstcopy(c, u):
            return pltpu.make_async_copy(dst_hbm.at[wid, c], dst_v.at[u],
                                         dsem[u])

        def stage(c, u):
            dstcopy(c, u).start()
            gather(c, u).start()

        def process(c, u, last):
            gather(c, u).wait()
            dstcopy(c, u).wait()
            pltpu.sync_copy(rows[u], acc_sh.at[dst_v.at[u, 0]], add=True)
            if not last:
                @pl.when(c + 3 < ch)
                def _():
                    stage(c + 3, u)

        for u in range(min(3, ch)):
            stage(u, u)

        def group(g, carry):
            c = 3 * g
            process(c, 0, False)
            process(c + 1, 1, False)
            process(c + 2, 2, False)
            return carry

        lax.fori_loop(0, ch // 3, group, 0)
        for r in range(ch % 3):
            process(ch - (ch % 3) + r, r, True)
        plsc.subcore_barrier()
        pltpu.sync_copy(acc_sh.at[pl.ds(sid * rpt, rpt)],
                        out_hbm.at[cid, pl.ds(sid * rpt, rpt)])

    return body(h, src1, dst3, zeros_tile)


def _dense0_body(x_ref, w_ref, b_ref, o_ref):
    o_ref[...] = jax.nn.relu(
        jnp.dot(x_ref[...], w_ref[...], preferred_element_type=jnp.float32)
        + b_ref[...])


def _layer_body(p_ref, x0_ref, h_ref, w_ref, o_ref, *, alpha, beta, eps):
    n = x0_ref.shape[0]
    agg = (p_ref[0] + p_ref[1])[:n]
    t = (1.0 - alpha) * agg + alpha * x0_ref[...]
    out = (1.0 - beta) * t + beta * jnp.dot(
        t, w_ref[...], preferred_element_type=jnp.float32)
    h2 = jax.nn.relu(out + h_ref[...])
    h2 = h2 - jnp.mean(h2, axis=0, keepdims=True)
    ms = jnp.mean(jnp.sum(h2 * h2, axis=-1))
    o_ref[...] = h2 / jnp.sqrt(eps + ms)


def _tail_body(h_ref, bat_ref, w1_ref, b1_ref, w2_ref, b2_ref, g_ref, be_ref,
               o_ref, *, ng, eps_bn):
    h = h_ref[...]
    bat = bat_ref[...]
    n = h.shape[0]
    gid = lax.broadcasted_iota(jnp.int32, (n, ng), 1)
    onehot = (bat == gid).astype(jnp.float32)
    gsum = lax.dot_general(onehot, h, (((0,), (0,)), ((), ())),
                           preferred_element_type=jnp.float32)
    cnt = jnp.sum(onehot, axis=0).reshape(ng, 1)
    gmax_rows = []
    for g in range(ng):
        m = bat == g
        gmax_rows.append(
            jnp.max(jnp.where(m, h, -jnp.inf), axis=0, keepdims=True))
    gmax = jnp.concatenate(gmax_rows, axis=0)
    cnt = jnp.maximum(cnt, 1.0)
    x2 = jnp.concatenate([gmax, gsum / cnt], axis=1)
    z = jax.nn.relu(
        jnp.dot(x2, w1_ref[...], preferred_element_type=jnp.float32)
        + b1_ref[...])
    mu = jnp.mean(z, axis=0, keepdims=True)
    var = jnp.mean((z - mu) ** 2, axis=0, keepdims=True)
    z = (z - mu) / jnp.sqrt(var + eps_bn) * g_ref[...] + be_ref[...]
    logits = (jnp.dot(z, w2_ref[...], preferred_element_type=jnp.float32)
              + b2_ref[...])
    mx = jnp.max(logits, axis=1, keepdims=True)
    lse = mx + jnp.log(jnp.sum(jnp.exp(logits - mx), axis=1, keepdims=True))
    o_ref[...] = logits - lse


def _tc_call(body, out_shape, *args):
    return pl.pallas_call(body, out_shape=out_shape)(*args)


def kernel(x, edge_index, batch, lin0_W, lin0_b, conv_W, lin1_W, lin1_b,
           lin2_W, lin2_b, bn_gamma, bn_beta):
    n, dim = x.shape
    hid = lin0_W.shape[1]
    e = edge_index.shape[1]

    assert e % _NW == 0, e
    epw = e // _NW
    k = 0
    for cand in range(128, 7, -1):
        if epw % cand or cand % 8:
            continue
        words = (-(-epw // 128) * 128 + 3 * 1024
                 + 3 * -(-cand // 8) * 8 * hid)
        if words <= 47000:
            k = cand
            break
    assert k, epw
    ch = epw // k
    rpt = -(-n // (_NS * 8)) * 8
    npad = rpt * _NS

    src1 = edge_index[0]
    dst3 = edge_index[1].reshape(_NW, ch, 1, k)
    zeros_tile = jnp.zeros((rpt, hid), dtype=jnp.float32)
    bat2 = batch.reshape(n, 1)

    x0 = _tc_call(_dense0_body, jax.ShapeDtypeStruct((n, hid), jnp.float32),
                  x, lin0_W, lin0_b.reshape(1, hid))
    h = x0
    for l in range(_NUM_LAYERS):
        beta = float(math.log(_THETA / (l + 1) + 1.0))
        partials = _sc_segment_sum(h, src1, dst3, zeros_tile,
                                   npad=npad, hid=hid, ch=ch, k=k)
        h = _tc_call(
            functools.partial(_layer_body, alpha=_ALPHA, beta=beta,
                              eps=_EPS_PN),
            jax.ShapeDtypeStruct((n, hid), jnp.float32),
            partials, x0, h, conv_W[l])

    out = _tc_call(
        functools.partial(_tail_body, ng=_NUM_GRAPHS, eps_bn=_EPS_BN),
        jax.ShapeDtypeStruct((_NUM_GRAPHS, lin2_W.shape[1]), jnp.float32),
        h, bat2, lin1_W, lin1_b.reshape(1, -1), lin2_W, lin2_b.reshape(1, -1),
        bn_gamma.reshape(1, -1), bn_beta.reshape(1, -1))
    return out
